# baseline (speedup 1.0000x reference)
import functools

import jax
import jax.numpy as jnp
from jax import lax
from jax.experimental import pallas as pl
from jax.experimental.pallas import tpu as pltpu
from jax.experimental.pallas import tpu_sc as plsc

BATCH = 16384
RANK = 64
PAIR = 2 * RANK
NUM_CORES = 2
NUM_SUBCORES = 16
NUM_WORKERS = NUM_CORES * NUM_SUBCORES
B_PER_W = BATCH // NUM_WORKERS
PASS_ROWS = 256
LANES = 16
TC_BLOCK = 2048

NUM_U = 100001
NUM_V = 1000001
N2_U = 51200
N2_V = 501760


def _pair_body(top_ref, bot_ref, out_ref):
    eye = jnp.eye(RANK, dtype=jnp.float32)
    dims = (((0,), (0,)), ((), ()))
    top = lax.dot_general(top_ref[...], eye, dims,
                          preferred_element_type=jnp.float32)
    bot = lax.dot_general(bot_ref[...], eye, dims,
                          preferred_element_type=jnp.float32)
    out_ref[...] = jnp.concatenate([top, bot], axis=1)


def _tc_pair_transpose(tableT, n2):
    k, n = tableT.shape
    grid = n2 // TC_BLOCK
    max_block = (n - 1) // TC_BLOCK

    def top_map(g):
        return (0, g)

    def bot_map(g):
        return (0, jnp.minimum(g + grid, max_block))

    return pl.pallas_call(
        _pair_body,
        grid=(grid,),
        in_specs=[
            pl.BlockSpec((k, TC_BLOCK), top_map),
            pl.BlockSpec((k, TC_BLOCK), bot_map),
        ],
        out_specs=pl.BlockSpec((TC_BLOCK, PAIR), lambda g: (g, 0)),
        out_shape=jax.ShapeDtypeStruct((n2, PAIR), jnp.float32),
    )(tableT, tableT)


def _sc_body(users_hbm, items_hbm, pred_hbm, u2_hbm, v2_hbm, out_hbm,
             uidxA, uidxB, iidxA, iidxB, ucol, icol, urows, vrows,
             pvec, outv, accv, sem_u, sem_v):
    wid = lax.axis_index("s") * NUM_CORES + lax.axis_index("c")
    base = wid * B_PER_W

    pltpu.sync_copy(users_hbm.at[pl.ds(base, PASS_ROWS)], uidxA)
    pltpu.sync_copy(users_hbm.at[pl.ds(base + PASS_ROWS, PASS_ROWS)], uidxB)
    pltpu.sync_copy(items_hbm.at[pl.ds(base, PASS_ROWS)], iidxA)
    pltpu.sync_copy(items_hbm.at[pl.ds(base + PASS_ROWS, PASS_ROWS)], iidxB)
    pltpu.sync_copy(pred_hbm, pvec)

    def make_fix(idx_ref, col_ref, col_off, n2):
        def fix(i, carry):
            r = idx_ref[pl.ds(i * LANES, LANES)]
            hi = (r >= n2).astype(jnp.int32)
            idx_ref[pl.ds(i * LANES, LANES)] = r - hi * n2
            col_ref[pl.ds(col_off + i * LANES, LANES)] = hi * RANK
            return carry
        return fix

    n_fix = PASS_ROWS // LANES
    lax.fori_loop(0, n_fix, make_fix(uidxA, ucol, 0, N2_U), 0)
    lax.fori_loop(0, n_fix, make_fix(uidxB, ucol, PASS_ROWS, N2_U), 0)
    lax.fori_loop(0, n_fix, make_fix(iidxA, icol, 0, N2_V), 0)
    lax.fori_loop(0, n_fix, make_fix(iidxB, icol, PASS_ROWS, N2_V), 0)

    p0 = pvec[pl.ds(0, LANES)]
    p1 = pvec[pl.ds(LANES, LANES)]
    p2 = pvec[pl.ds(2 * LANES, LANES)]
    p3 = pvec[pl.ds(3 * LANES, LANES)]

    lane_ids = lax.iota(jnp.int32, LANES)
    col_base = lane_ids * LANES
    chunks = [lane_ids + c * LANES for c in range(4)]
    pchunks = [p0, p1, p2, p3]

    def do_pass(pbase, uref, iref):
        cu = pltpu.async_copy(u2_hbm.at[uref], urows, sem_u)
        cv = pltpu.async_copy(v2_hbm.at[iref], vrows, sem_v)
        cu.wait()
        cv.wait()

        def group(g, carry):
            b0 = g * LANES
            for j in range(LANES):
                b = b0 + j
                bsplat = jnp.full((LANES,), b, jnp.int32)
                uco = plsc.load_gather(ucol, [bsplat + pbase])
                ico = plsc.load_gather(icol, [bsplat + pbase])
                acc = jnp.zeros((LANES,), jnp.float32)
                for c in range(4):
                    gu = plsc.load_gather(urows, [bsplat, uco + chunks[c]])
                    gv = plsc.load_gather(vrows, [bsplat, ico + chunks[c]])
                    acc += gu * gv * pchunks[c]
                accv[pl.ds(j * LANES, LANES)] = acc
            vec = plsc.load_gather(accv, [col_base])
            for k in range(1, LANES):
                vec += plsc.load_gather(accv, [col_base + k])
            outv[pl.ds(pbase + b0, LANES)] = vec
            return carry

        lax.fori_loop(0, PASS_ROWS // LANES, group, 0)

    do_pass(0, uidxA, iidxA)
    do_pass(PASS_ROWS, uidxB, iidxB)

    pltpu.sync_copy(outv, out_hbm.at[pl.ds(base, B_PER_W)])


@functools.partial(
    pl.kernel,
    mesh=plsc.VectorSubcoreMesh(core_axis_name="c", subcore_axis_name="s"),
    out_type=jax.ShapeDtypeStruct((BATCH,), jnp.float32),
    compiler_params=pltpu.CompilerParams(
        needs_layout_passes=False, use_tc_tiling_on_sc=True),
    scratch_types=[
        pltpu.VMEM((PASS_ROWS,), jnp.int32),
        pltpu.VMEM((PASS_ROWS,), jnp.int32),
        pltpu.VMEM((PASS_ROWS,), jnp.int32),
        pltpu.VMEM((PASS_ROWS,), jnp.int32),
        pltpu.VMEM((B_PER_W,), jnp.int32),
        pltpu.VMEM((B_PER_W,), jnp.int32),
        pltpu.VMEM((PASS_ROWS, PAIR), jnp.float32),
        pltpu.VMEM((PASS_ROWS, PAIR), jnp.float32),
        pltpu.VMEM((RANK,), jnp.float32),
        pltpu.VMEM((B_PER_W,), jnp.float32),
        pltpu.VMEM((LANES * LANES,), jnp.float32),
        pltpu.SemaphoreType.DMA,
        pltpu.SemaphoreType.DMA,
    ],
)
def _sc_kernel(users_hbm, items_hbm, pred_hbm, u2_hbm, v2_hbm, out_hbm,
               uidxA, uidxB, iidxA, iidxB, ucol, icol, urows, vrows,
               pvec, outv, accv, sem_u, sem_v):
    _sc_body(users_hbm, items_hbm, pred_hbm, u2_hbm, v2_hbm, out_hbm,
             uidxA, uidxB, iidxA, iidxB, ucol, icol, urows, vrows,
             pvec, outv, accv, sem_u, sem_v)


def kernel(users, items, embed_U, embed_V, predict_layer):
    pred = predict_layer.reshape(RANK)
    u2 = _tc_pair_transpose(embed_U.T, N2_U)
    v2 = _tc_pair_transpose(embed_V.T, N2_V)
    return _sc_kernel(users, items, pred, u2, v2)


# speedup vs baseline: 1.2279x; 1.2279x over previous
import functools

import jax
import jax.numpy as jnp
from jax import lax
from jax.experimental import pallas as pl
from jax.experimental.pallas import tpu as pltpu
from jax.experimental.pallas import tpu_sc as plsc

BATCH = 16384
RANK = 64
PAIR = 2 * RANK
NUM_CORES = 2
NUM_SUBCORES = 16
NUM_WORKERS = NUM_CORES * NUM_SUBCORES
B_PER_W = BATCH // NUM_WORKERS
PASS_ROWS = 256
LANES = 16
TC_BLOCK = 4096

NUM_U = 100001
NUM_V = 1000001
N2_U = 53248
N2_V = 503808


def _pair_body(top_ref, bot_ref, out_ref):
    out_ref[:, 0:RANK] = top_ref[...].T
    out_ref[:, RANK:PAIR] = bot_ref[...].T


def _tc_pair_transpose(tableT, n2):
    k, n = tableT.shape
    grid = n2 // TC_BLOCK
    max_block = (n - 1) // TC_BLOCK

    def top_map(g):
        return (0, g)

    def bot_map(g):
        return (0, jnp.minimum(g + grid, max_block))

    return pl.pallas_call(
        _pair_body,
        grid=(grid,),
        in_specs=[
            pl.BlockSpec((k, TC_BLOCK), top_map),
            pl.BlockSpec((k, TC_BLOCK), bot_map),
        ],
        out_specs=pl.BlockSpec((TC_BLOCK, PAIR), lambda g: (g, 0)),
        out_shape=jax.ShapeDtypeStruct((n2, PAIR), jnp.float32),
        compiler_params=pltpu.CompilerParams(
            fuse_transposed_lhs_in_matmul=True),
    )(tableT, tableT)


def _sc_body(users_hbm, items_hbm, pred_hbm, u2_hbm, v2_hbm, out_hbm,
             uidxA, uidxB, iidxA, iidxB, ucol, icol, urows, vrows,
             pvec, outv, accv, sem_u, sem_v):
    wid = lax.axis_index("s") * NUM_CORES + lax.axis_index("c")
    base = wid * B_PER_W

    pltpu.sync_copy(users_hbm.at[pl.ds(base, PASS_ROWS)], uidxA)
    pltpu.sync_copy(users_hbm.at[pl.ds(base + PASS_ROWS, PASS_ROWS)], uidxB)
    pltpu.sync_copy(items_hbm.at[pl.ds(base, PASS_ROWS)], iidxA)
    pltpu.sync_copy(items_hbm.at[pl.ds(base + PASS_ROWS, PASS_ROWS)], iidxB)
    pltpu.sync_copy(pred_hbm, pvec)

    def make_fix(idx_ref, col_ref, col_off, n2):
        def fix(i, carry):
            r = idx_ref[pl.ds(i * LANES, LANES)]
            hi = (r >= n2).astype(jnp.int32)
            idx_ref[pl.ds(i * LANES, LANES)] = r - hi * n2
            col_ref[pl.ds(col_off + i * LANES, LANES)] = hi * RANK
            return carry
        return fix

    n_fix = PASS_ROWS // LANES
    lax.fori_loop(0, n_fix, make_fix(uidxA, ucol, 0, N2_U), 0)
    lax.fori_loop(0, n_fix, make_fix(uidxB, ucol, PASS_ROWS, N2_U), 0)
    lax.fori_loop(0, n_fix, make_fix(iidxA, icol, 0, N2_V), 0)
    lax.fori_loop(0, n_fix, make_fix(iidxB, icol, PASS_ROWS, N2_V), 0)

    p0 = pvec[pl.ds(0, LANES)]
    p1 = pvec[pl.ds(LANES, LANES)]
    p2 = pvec[pl.ds(2 * LANES, LANES)]
    p3 = pvec[pl.ds(3 * LANES, LANES)]

    lane_ids = lax.iota(jnp.int32, LANES)
    col_base = lane_ids * LANES
    chunks = [lane_ids + c * LANES for c in range(4)]
    pchunks = [p0, p1, p2, p3]

    def do_pass(pbase, uref, iref):
        cu = pltpu.async_copy(u2_hbm.at[uref], urows, sem_u)
        cv = pltpu.async_copy(v2_hbm.at[iref], vrows, sem_v)
        cu.wait()
        cv.wait()

        def group(g, carry):
            b0 = g * LANES
            for j in range(LANES):
                b = b0 + j
                bsplat = jnp.full((LANES,), b, jnp.int32)
                uco = plsc.load_gather(ucol, [bsplat + pbase])
                ico = plsc.load_gather(icol, [bsplat + pbase])
                acc = jnp.zeros((LANES,), jnp.float32)
                for c in range(4):
                    gu = plsc.load_gather(urows, [bsplat, uco + chunks[c]])
                    gv = plsc.load_gather(vrows, [bsplat, ico + chunks[c]])
                    acc += gu * gv * pchunks[c]
                accv[pl.ds(j * LANES, LANES)] = acc
            vec = plsc.load_gather(accv, [col_base])
            for k in range(1, LANES):
                vec += plsc.load_gather(accv, [col_base + k])
            outv[pl.ds(pbase + b0, LANES)] = vec
            return carry

        lax.fori_loop(0, PASS_ROWS // LANES, group, 0)

    do_pass(0, uidxA, iidxA)
    do_pass(PASS_ROWS, uidxB, iidxB)

    pltpu.sync_copy(outv, out_hbm.at[pl.ds(base, B_PER_W)])


@functools.partial(
    pl.kernel,
    mesh=plsc.VectorSubcoreMesh(core_axis_name="c", subcore_axis_name="s"),
    out_type=jax.ShapeDtypeStruct((BATCH,), jnp.float32),
    compiler_params=pltpu.CompilerParams(
        needs_layout_passes=False, use_tc_tiling_on_sc=True),
    scratch_types=[
        pltpu.VMEM((PASS_ROWS,), jnp.int32),
        pltpu.VMEM((PASS_ROWS,), jnp.int32),
        pltpu.VMEM((PASS_ROWS,), jnp.int32),
        pltpu.VMEM((PASS_ROWS,), jnp.int32),
        pltpu.VMEM((B_PER_W,), jnp.int32),
        pltpu.VMEM((B_PER_W,), jnp.int32),
        pltpu.VMEM((PASS_ROWS, PAIR), jnp.float32),
        pltpu.VMEM((PASS_ROWS, PAIR), jnp.float32),
        pltpu.VMEM((RANK,), jnp.float32),
        pltpu.VMEM((B_PER_W,), jnp.float32),
        pltpu.VMEM((LANES * LANES,), jnp.float32),
        pltpu.SemaphoreType.DMA,
        pltpu.SemaphoreType.DMA,
    ],
)
def _sc_kernel(users_hbm, items_hbm, pred_hbm, u2_hbm, v2_hbm, out_hbm,
               uidxA, uidxB, iidxA, iidxB, ucol, icol, urows, vrows,
               pvec, outv, accv, sem_u, sem_v):
    _sc_body(users_hbm, items_hbm, pred_hbm, u2_hbm, v2_hbm, out_hbm,
             uidxA, uidxB, iidxA, iidxB, ucol, icol, urows, vrows,
             pvec, outv, accv, sem_u, sem_v)


def kernel(users, items, embed_U, embed_V, predict_layer):
    pred = predict_layer.reshape(RANK)
    u2 = _tc_pair_transpose(embed_U.T, N2_U)
    v2 = _tc_pair_transpose(embed_V.T, N2_V)
    return _sc_kernel(users, items, pred, u2, v2)


# speedup vs baseline: 1.3660x; 1.1124x over previous
import functools

import jax
import jax.numpy as jnp
from jax import lax
from jax.experimental import pallas as pl
from jax.experimental.pallas import tpu as pltpu
from jax.experimental.pallas import tpu_sc as plsc

BATCH = 16384
RANK = 64
PAIR = 2 * RANK
NUM_CORES = 2
NUM_SUBCORES = 16
NUM_WORKERS = NUM_CORES * NUM_SUBCORES
B_PER_W = BATCH // NUM_WORKERS
PASS_ROWS = 256
LANES = 16
TC_BLOCK = 8192

NUM_U = 100001
NUM_V = 1000001
N2_U = 57344
N2_V = 507904


def _pair_body(top_ref, bot_ref, out_ref):
    out_ref[:, 0:RANK] = top_ref[...].T
    out_ref[:, RANK:PAIR] = bot_ref[...].T


def _tc_pair_transpose(tableT, n2):
    k, n = tableT.shape
    grid = n2 // TC_BLOCK
    max_block = (n - 1) // TC_BLOCK

    def top_map(g):
        return (0, g)

    def bot_map(g):
        return (0, jnp.minimum(g + grid, max_block))

    return pl.pallas_call(
        _pair_body,
        grid=(grid,),
        in_specs=[
            pl.BlockSpec((k, TC_BLOCK), top_map),
            pl.BlockSpec((k, TC_BLOCK), bot_map),
        ],
        out_specs=pl.BlockSpec((TC_BLOCK, PAIR), lambda g: (g, 0)),
        out_shape=jax.ShapeDtypeStruct((n2, PAIR), jnp.float32),
        compiler_params=pltpu.CompilerParams(
            fuse_transposed_lhs_in_matmul=True),
    )(tableT, tableT)


def _sc_body(users_hbm, items_hbm, pred_hbm, u2_hbm, v2_hbm, out_hbm,
             uidxA, uidxB, iidxA, iidxB, ucol, icol, urows, vrows,
             pvec, outv, accv, sem_u, sem_v):
    wid = lax.axis_index("s") * NUM_CORES + lax.axis_index("c")
    base = wid * B_PER_W

    pltpu.sync_copy(users_hbm.at[pl.ds(base, PASS_ROWS)], uidxA)
    pltpu.sync_copy(users_hbm.at[pl.ds(base + PASS_ROWS, PASS_ROWS)], uidxB)
    pltpu.sync_copy(items_hbm.at[pl.ds(base, PASS_ROWS)], iidxA)
    pltpu.sync_copy(items_hbm.at[pl.ds(base + PASS_ROWS, PASS_ROWS)], iidxB)
    pltpu.sync_copy(pred_hbm, pvec)

    def make_fix(idx_ref, col_ref, col_off, n2):
        def fix(i, carry):
            r = idx_ref[pl.ds(i * LANES, LANES)]
            hi = (r >= n2).astype(jnp.int32)
            idx_ref[pl.ds(i * LANES, LANES)] = r - hi * n2
            col_ref[pl.ds(col_off + i * LANES, LANES)] = hi * RANK
            return carry
        return fix

    n_fix = PASS_ROWS // LANES
    lax.fori_loop(0, n_fix, make_fix(uidxA, ucol, 0, N2_U), 0)
    lax.fori_loop(0, n_fix, make_fix(uidxB, ucol, PASS_ROWS, N2_U), 0)
    lax.fori_loop(0, n_fix, make_fix(iidxA, icol, 0, N2_V), 0)
    lax.fori_loop(0, n_fix, make_fix(iidxB, icol, PASS_ROWS, N2_V), 0)

    p0 = pvec[pl.ds(0, LANES)]
    p1 = pvec[pl.ds(LANES, LANES)]
    p2 = pvec[pl.ds(2 * LANES, LANES)]
    p3 = pvec[pl.ds(3 * LANES, LANES)]

    lane_ids = lax.iota(jnp.int32, LANES)
    col_base = lane_ids * LANES
    chunks = [lane_ids + c * LANES for c in range(4)]
    pchunks = [p0, p1, p2, p3]

    def do_pass(pbase, uref, iref):
        cu = pltpu.async_copy(u2_hbm.at[uref], urows, sem_u)
        cv = pltpu.async_copy(v2_hbm.at[iref], vrows, sem_v)
        cu.wait()
        cv.wait()

        def group(g, carry):
            b0 = g * LANES
            for j in range(LANES):
                b = b0 + j
                bsplat = jnp.full((LANES,), b, jnp.int32)
                uco = plsc.load_gather(ucol, [bsplat + pbase])
                ico = plsc.load_gather(icol, [bsplat + pbase])
                acc = jnp.zeros((LANES,), jnp.float32)
                for c in range(4):
                    gu = plsc.load_gather(urows, [bsplat, uco + chunks[c]])
                    gv = plsc.load_gather(vrows, [bsplat, ico + chunks[c]])
                    acc += gu * gv * pchunks[c]
                accv[pl.ds(j * LANES, LANES)] = acc
            vec = plsc.load_gather(accv, [col_base])
            for k in range(1, LANES):
                vec += plsc.load_gather(accv, [col_base + k])
            outv[pl.ds(pbase + b0, LANES)] = vec
            return carry

        lax.fori_loop(0, PASS_ROWS // LANES, group, 0)

    do_pass(0, uidxA, iidxA)
    do_pass(PASS_ROWS, uidxB, iidxB)

    pltpu.sync_copy(outv, out_hbm.at[pl.ds(base, B_PER_W)])


@functools.partial(
    pl.kernel,
    mesh=plsc.VectorSubcoreMesh(core_axis_name="c", subcore_axis_name="s"),
    out_type=jax.ShapeDtypeStruct((BATCH,), jnp.float32),
    compiler_params=pltpu.CompilerParams(
        needs_layout_passes=False, use_tc_tiling_on_sc=True),
    scratch_types=[
        pltpu.VMEM((PASS_ROWS,), jnp.int32),
        pltpu.VMEM((PASS_ROWS,), jnp.int32),
        pltpu.VMEM((PASS_ROWS,), jnp.int32),
        pltpu.VMEM((PASS_ROWS,), jnp.int32),
        pltpu.VMEM((B_PER_W,), jnp.int32),
        pltpu.VMEM((B_PER_W,), jnp.int32),
        pltpu.VMEM((PASS_ROWS, PAIR), jnp.float32),
        pltpu.VMEM((PASS_ROWS, PAIR), jnp.float32),
        pltpu.VMEM((RANK,), jnp.float32),
        pltpu.VMEM((B_PER_W,), jnp.float32),
        pltpu.VMEM((LANES * LANES,), jnp.float32),
        pltpu.SemaphoreType.DMA,
        pltpu.SemaphoreType.DMA,
    ],
)
def _sc_kernel(users_hbm, items_hbm, pred_hbm, u2_hbm, v2_hbm, out_hbm,
               uidxA, uidxB, iidxA, iidxB, ucol, icol, urows, vrows,
               pvec, outv, accv, sem_u, sem_v):
    _sc_body(users_hbm, items_hbm, pred_hbm, u2_hbm, v2_hbm, out_hbm,
             uidxA, uidxB, iidxA, iidxB, ucol, icol, urows, vrows,
             pvec, outv, accv, sem_u, sem_v)


def kernel(users, items, embed_U, embed_V, predict_layer):
    pred = predict_layer.reshape(RANK)
    u2 = _tc_pair_transpose(embed_U.T, N2_U)
    v2 = _tc_pair_transpose(embed_V.T, N2_V)
    return _sc_kernel(users, items, pred, u2, v2)


# speedup vs baseline: 1.4114x; 1.0333x over previous
import functools

import jax
import jax.numpy as jnp
from jax import lax
from jax.experimental import pallas as pl
from jax.experimental.pallas import tpu as pltpu
from jax.experimental.pallas import tpu_sc as plsc

BATCH = 16384
RANK = 64
PAIR = 2 * RANK
NUM_CORES = 2
NUM_SUBCORES = 16
NUM_WORKERS = NUM_CORES * NUM_SUBCORES
B_PER_W = BATCH // NUM_WORKERS
PASS_ROWS = 256
LANES = 16
TC_BLOCK = 16384

NUM_U = 100001
NUM_V = 1000001
N2_U = 65536
N2_V = 507904


def _pair_body(top_ref, bot_ref, out_ref):
    out_ref[:, 0:RANK] = top_ref[...].T
    out_ref[:, RANK:PAIR] = bot_ref[...].T


def _tc_pair_transpose(tableT, n2):
    k, n = tableT.shape
    grid = n2 // TC_BLOCK
    max_block = (n - 1) // TC_BLOCK

    def top_map(g):
        return (0, g)

    def bot_map(g):
        return (0, jnp.minimum(g + grid, max_block))

    return pl.pallas_call(
        _pair_body,
        grid=(grid,),
        in_specs=[
            pl.BlockSpec((k, TC_BLOCK), top_map),
            pl.BlockSpec((k, TC_BLOCK), bot_map),
        ],
        out_specs=pl.BlockSpec((TC_BLOCK, PAIR), lambda g: (g, 0)),
        out_shape=jax.ShapeDtypeStruct((n2, PAIR), jnp.float32),
        compiler_params=pltpu.CompilerParams(
            fuse_transposed_lhs_in_matmul=True),
    )(tableT, tableT)


def _sc_body(users_hbm, items_hbm, pred_hbm, u2_hbm, v2_hbm, out_hbm,
             uidxA, uidxB, iidxA, iidxB, ucol, icol, urows, vrows,
             pvec, outv, accv, sem_u, sem_v):
    wid = lax.axis_index("s") * NUM_CORES + lax.axis_index("c")
    base = wid * B_PER_W

    pltpu.sync_copy(users_hbm.at[pl.ds(base, PASS_ROWS)], uidxA)
    pltpu.sync_copy(users_hbm.at[pl.ds(base + PASS_ROWS, PASS_ROWS)], uidxB)
    pltpu.sync_copy(items_hbm.at[pl.ds(base, PASS_ROWS)], iidxA)
    pltpu.sync_copy(items_hbm.at[pl.ds(base + PASS_ROWS, PASS_ROWS)], iidxB)
    pltpu.sync_copy(pred_hbm, pvec)

    def make_fix(idx_ref, col_ref, col_off, n2):
        def fix(i, carry):
            r = idx_ref[pl.ds(i * LANES, LANES)]
            hi = (r >= n2).astype(jnp.int32)
            idx_ref[pl.ds(i * LANES, LANES)] = r - hi * n2
            col_ref[pl.ds(col_off + i * LANES, LANES)] = hi * RANK
            return carry
        return fix

    n_fix = PASS_ROWS // LANES
    lax.fori_loop(0, n_fix, make_fix(uidxA, ucol, 0, N2_U), 0)
    lax.fori_loop(0, n_fix, make_fix(uidxB, ucol, PASS_ROWS, N2_U), 0)
    lax.fori_loop(0, n_fix, make_fix(iidxA, icol, 0, N2_V), 0)
    lax.fori_loop(0, n_fix, make_fix(iidxB, icol, PASS_ROWS, N2_V), 0)

    p0 = pvec[pl.ds(0, LANES)]
    p1 = pvec[pl.ds(LANES, LANES)]
    p2 = pvec[pl.ds(2 * LANES, LANES)]
    p3 = pvec[pl.ds(3 * LANES, LANES)]

    lane_ids = lax.iota(jnp.int32, LANES)
    col_base = lane_ids * LANES
    chunks = [lane_ids + c * LANES for c in range(4)]
    pchunks = [p0, p1, p2, p3]

    def do_pass(pbase, uref, iref):
        cu = pltpu.async_copy(u2_hbm.at[uref], urows, sem_u)
        cv = pltpu.async_copy(v2_hbm.at[iref], vrows, sem_v)
        cu.wait()
        cv.wait()

        def group(g, carry):
            b0 = g * LANES
            for j in range(LANES):
                b = b0 + j
                bsplat = jnp.full((LANES,), b, jnp.int32)
                uco = plsc.load_gather(ucol, [bsplat + pbase])
                ico = plsc.load_gather(icol, [bsplat + pbase])
                acc = jnp.zeros((LANES,), jnp.float32)
                for c in range(4):
                    gu = plsc.load_gather(urows, [bsplat, uco + chunks[c]])
                    gv = plsc.load_gather(vrows, [bsplat, ico + chunks[c]])
                    acc += gu * gv * pchunks[c]
                accv[pl.ds(j * LANES, LANES)] = acc
            vec = plsc.load_gather(accv, [col_base])
            for k in range(1, LANES):
                vec += plsc.load_gather(accv, [col_base + k])
            outv[pl.ds(pbase + b0, LANES)] = vec
            return carry

        lax.fori_loop(0, PASS_ROWS // LANES, group, 0)

    do_pass(0, uidxA, iidxA)
    do_pass(PASS_ROWS, uidxB, iidxB)

    pltpu.sync_copy(outv, out_hbm.at[pl.ds(base, B_PER_W)])


@functools.partial(
    pl.kernel,
    mesh=plsc.VectorSubcoreMesh(core_axis_name="c", subcore_axis_name="s"),
    out_type=jax.ShapeDtypeStruct((BATCH,), jnp.float32),
    compiler_params=pltpu.CompilerParams(
        needs_layout_passes=False, use_tc_tiling_on_sc=True),
    scratch_types=[
        pltpu.VMEM((PASS_ROWS,), jnp.int32),
        pltpu.VMEM((PASS_ROWS,), jnp.int32),
        pltpu.VMEM((PASS_ROWS,), jnp.int32),
        pltpu.VMEM((PASS_ROWS,), jnp.int32),
        pltpu.VMEM((B_PER_W,), jnp.int32),
        pltpu.VMEM((B_PER_W,), jnp.int32),
        pltpu.VMEM((PASS_ROWS, PAIR), jnp.float32),
        pltpu.VMEM((PASS_ROWS, PAIR), jnp.float32),
        pltpu.VMEM((RANK,), jnp.float32),
        pltpu.VMEM((B_PER_W,), jnp.float32),
        pltpu.VMEM((LANES * LANES,), jnp.float32),
        pltpu.SemaphoreType.DMA,
        pltpu.SemaphoreType.DMA,
    ],
)
def _sc_kernel(users_hbm, items_hbm, pred_hbm, u2_hbm, v2_hbm, out_hbm,
               uidxA, uidxB, iidxA, iidxB, ucol, icol, urows, vrows,
               pvec, outv, accv, sem_u, sem_v):
    _sc_body(users_hbm, items_hbm, pred_hbm, u2_hbm, v2_hbm, out_hbm,
             uidxA, uidxB, iidxA, iidxB, ucol, icol, urows, vrows,
             pvec, outv, accv, sem_u, sem_v)


def kernel(users, items, embed_U, embed_V, predict_layer):
    pred = predict_layer.reshape(RANK)
    u2 = _tc_pair_transpose(embed_U.T, N2_U)
    v2 = _tc_pair_transpose(embed_V.T, N2_V)
    return _sc_kernel(users, items, pred, u2, v2)


# speedup vs baseline: 1.7651x; 1.2506x over previous
import functools

import jax
import jax.numpy as jnp
from jax import lax
from jax.experimental import pallas as pl
from jax.experimental.pallas import tpu as pltpu
from jax.experimental.pallas import tpu_sc as plsc

BATCH = 16384
RANK = 64
PAIR = 2 * RANK
NUM_CORES = 2
NUM_SUBCORES = 16
NUM_WORKERS = NUM_CORES * NUM_SUBCORES
B_PER_W = BATCH // NUM_WORKERS
PASS_ROWS = 256
LANES = 16

BLK = 16384
HBLK = BLK // 2
BLK_SHIFT = 14
HALF_SHIFT = 13
HALF_MASK = HBLK - 1

NUM_U = 100001
NUM_V = 1000001


def _pack_body(src_ref, out_ref):
    x = src_ref[...]
    y = jnp.concatenate([x[:, :HBLK], x[:, HBLK:]], axis=0)
    out_ref[...] = y.T


def _tc_pack(tableT):
    k, n = tableT.shape
    grid = (n + BLK - 1) // BLK
    return pl.pallas_call(
        _pack_body,
        grid=(grid,),
        in_specs=[pl.BlockSpec((k, BLK), lambda g: (0, g))],
        out_specs=pl.BlockSpec((HBLK, PAIR), lambda g: (g, 0)),
        out_shape=jax.ShapeDtypeStruct((grid * HBLK, PAIR), jnp.float32),
    )(tableT)


def _sc_body(users_hbm, items_hbm, pred_hbm, u2_hbm, v2_hbm, out_hbm,
             uidxA, uidxB, iidxA, iidxB, ucol, icol, urows, vrows,
             pvec, outv, accv, sem_u, sem_v):
    wid = lax.axis_index("s") * NUM_CORES + lax.axis_index("c")
    base = wid * B_PER_W

    pltpu.sync_copy(users_hbm.at[pl.ds(base, PASS_ROWS)], uidxA)
    pltpu.sync_copy(users_hbm.at[pl.ds(base + PASS_ROWS, PASS_ROWS)], uidxB)
    pltpu.sync_copy(items_hbm.at[pl.ds(base, PASS_ROWS)], iidxA)
    pltpu.sync_copy(items_hbm.at[pl.ds(base + PASS_ROWS, PASS_ROWS)], iidxB)
    pltpu.sync_copy(pred_hbm, pvec)

    def make_fix(idx_ref, col_ref, col_off):
        def fix(i, carry):
            r = idx_ref[pl.ds(i * LANES, LANES)]
            slot = ((r >> BLK_SHIFT) << (BLK_SHIFT - 1)) + (r & HALF_MASK)
            half = (r >> HALF_SHIFT) & 1
            idx_ref[pl.ds(i * LANES, LANES)] = slot
            col_ref[pl.ds(col_off + i * LANES, LANES)] = half * RANK
            return carry
        return fix

    n_fix = PASS_ROWS // LANES
    lax.fori_loop(0, n_fix, make_fix(uidxA, ucol, 0), 0)
    lax.fori_loop(0, n_fix, make_fix(uidxB, ucol, PASS_ROWS), 0)
    lax.fori_loop(0, n_fix, make_fix(iidxA, icol, 0), 0)
    lax.fori_loop(0, n_fix, make_fix(iidxB, icol, PASS_ROWS), 0)

    p0 = pvec[pl.ds(0, LANES)]
    p1 = pvec[pl.ds(LANES, LANES)]
    p2 = pvec[pl.ds(2 * LANES, LANES)]
    p3 = pvec[pl.ds(3 * LANES, LANES)]

    lane_ids = lax.iota(jnp.int32, LANES)
    col_base = lane_ids * LANES
    chunks = [lane_ids + c * LANES for c in range(4)]
    pchunks = [p0, p1, p2, p3]

    def do_pass(pbase, uref, iref):
        cu = pltpu.async_copy(u2_hbm.at[uref], urows, sem_u)
        cv = pltpu.async_copy(v2_hbm.at[iref], vrows, sem_v)
        cu.wait()
        cv.wait()

        def group(g, carry):
            b0 = g * LANES
            for j in range(LANES):
                b = b0 + j
                bsplat = jnp.full((LANES,), b, jnp.int32)
                uco = plsc.load_gather(ucol, [bsplat + pbase])
                ico = plsc.load_gather(icol, [bsplat + pbase])
                acc = jnp.zeros((LANES,), jnp.float32)
                for c in range(4):
                    gu = plsc.load_gather(urows, [bsplat, uco + chunks[c]])
                    gv = plsc.load_gather(vrows, [bsplat, ico + chunks[c]])
                    acc += gu * gv * pchunks[c]
                accv[pl.ds(j * LANES, LANES)] = acc
            vec = plsc.load_gather(accv, [col_base])
            for k in range(1, LANES):
                vec += plsc.load_gather(accv, [col_base + k])
            outv[pl.ds(pbase + b0, LANES)] = vec
            return carry

        lax.fori_loop(0, PASS_ROWS // LANES, group, 0)

    do_pass(0, uidxA, iidxA)
    do_pass(PASS_ROWS, uidxB, iidxB)

    pltpu.sync_copy(outv, out_hbm.at[pl.ds(base, B_PER_W)])


@functools.partial(
    pl.kernel,
    mesh=plsc.VectorSubcoreMesh(core_axis_name="c", subcore_axis_name="s"),
    out_type=jax.ShapeDtypeStruct((BATCH,), jnp.float32),
    compiler_params=pltpu.CompilerParams(
        needs_layout_passes=False, use_tc_tiling_on_sc=True),
    scratch_types=[
        pltpu.VMEM((PASS_ROWS,), jnp.int32),
        pltpu.VMEM((PASS_ROWS,), jnp.int32),
        pltpu.VMEM((PASS_ROWS,), jnp.int32),
        pltpu.VMEM((PASS_ROWS,), jnp.int32),
        pltpu.VMEM((B_PER_W,), jnp.int32),
        pltpu.VMEM((B_PER_W,), jnp.int32),
        pltpu.VMEM((PASS_ROWS, PAIR), jnp.float32),
        pltpu.VMEM((PASS_ROWS, PAIR), jnp.float32),
        pltpu.VMEM((RANK,), jnp.float32),
        pltpu.VMEM((B_PER_W,), jnp.float32),
        pltpu.VMEM((LANES * LANES,), jnp.float32),
        pltpu.SemaphoreType.DMA,
        pltpu.SemaphoreType.DMA,
    ],
)
def _sc_kernel(users_hbm, items_hbm, pred_hbm, u2_hbm, v2_hbm, out_hbm,
               uidxA, uidxB, iidxA, iidxB, ucol, icol, urows, vrows,
               pvec, outv, accv, sem_u, sem_v):
    _sc_body(users_hbm, items_hbm, pred_hbm, u2_hbm, v2_hbm, out_hbm,
             uidxA, uidxB, iidxA, iidxB, ucol, icol, urows, vrows,
             pvec, outv, accv, sem_u, sem_v)


def kernel(users, items, embed_U, embed_V, predict_layer):
    pred = predict_layer.reshape(RANK)
    u2 = _tc_pack(embed_U.T)
    v2 = _tc_pack(embed_V.T)
    return _sc_kernel(users, items, pred, u2, v2)


# speedup vs baseline: 1.7955x; 1.0172x over previous
import functools

import jax
import jax.numpy as jnp
from jax import lax
from jax.experimental import pallas as pl
from jax.experimental.pallas import tpu as pltpu
from jax.experimental.pallas import tpu_sc as plsc

BATCH = 16384
RANK = 64
PAIR = 2 * RANK
NUM_CORES = 2
NUM_SUBCORES = 16
NUM_WORKERS = NUM_CORES * NUM_SUBCORES
B_PER_W = BATCH // NUM_WORKERS
PASS_ROWS = 256
LANES = 16

BLK = 32768
HBLK = BLK // 2
BLK_SHIFT = 15
HALF_SHIFT = 14
HALF_MASK = HBLK - 1

NUM_U = 100001
NUM_V = 1000001


def _pack_body(src_ref, out_ref):
    x = src_ref[...]
    y = jnp.concatenate([x[:, :HBLK], x[:, HBLK:]], axis=0)
    out_ref[...] = y.T


def _tc_pack(tableT):
    k, n = tableT.shape
    grid = (n + BLK - 1) // BLK
    return pl.pallas_call(
        _pack_body,
        grid=(grid,),
        in_specs=[pl.BlockSpec((k, BLK), lambda g: (0, g))],
        out_specs=pl.BlockSpec((HBLK, PAIR), lambda g: (g, 0)),
        out_shape=jax.ShapeDtypeStruct((grid * HBLK, PAIR), jnp.float32),
    )(tableT)


def _sc_body(users_hbm, items_hbm, pred_hbm, u2_hbm, v2_hbm, out_hbm,
             uidxA, uidxB, iidxA, iidxB, ucol, icol, urows, vrows,
             pvec, outv, accv, sem_u, sem_v):
    wid = lax.axis_index("s") * NUM_CORES + lax.axis_index("c")
    base = wid * B_PER_W

    pltpu.sync_copy(users_hbm.at[pl.ds(base, PASS_ROWS)], uidxA)
    pltpu.sync_copy(users_hbm.at[pl.ds(base + PASS_ROWS, PASS_ROWS)], uidxB)
    pltpu.sync_copy(items_hbm.at[pl.ds(base, PASS_ROWS)], iidxA)
    pltpu.sync_copy(items_hbm.at[pl.ds(base + PASS_ROWS, PASS_ROWS)], iidxB)
    pltpu.sync_copy(pred_hbm, pvec)

    def make_fix(idx_ref, col_ref, col_off):
        def fix(i, carry):
            r = idx_ref[pl.ds(i * LANES, LANES)]
            slot = ((r >> BLK_SHIFT) << (BLK_SHIFT - 1)) + (r & HALF_MASK)
            half = (r >> HALF_SHIFT) & 1
            idx_ref[pl.ds(i * LANES, LANES)] = slot
            col_ref[pl.ds(col_off + i * LANES, LANES)] = half * RANK
            return carry
        return fix

    n_fix = PASS_ROWS // LANES
    lax.fori_loop(0, n_fix, make_fix(uidxA, ucol, 0), 0)
    lax.fori_loop(0, n_fix, make_fix(uidxB, ucol, PASS_ROWS), 0)
    lax.fori_loop(0, n_fix, make_fix(iidxA, icol, 0), 0)
    lax.fori_loop(0, n_fix, make_fix(iidxB, icol, PASS_ROWS), 0)

    p0 = pvec[pl.ds(0, LANES)]
    p1 = pvec[pl.ds(LANES, LANES)]
    p2 = pvec[pl.ds(2 * LANES, LANES)]
    p3 = pvec[pl.ds(3 * LANES, LANES)]

    lane_ids = lax.iota(jnp.int32, LANES)
    col_base = lane_ids * LANES
    chunks = [lane_ids + c * LANES for c in range(4)]
    pchunks = [p0, p1, p2, p3]

    def do_pass(pbase, uref, iref):
        cu = pltpu.async_copy(u2_hbm.at[uref], urows, sem_u)
        cv = pltpu.async_copy(v2_hbm.at[iref], vrows, sem_v)
        cu.wait()
        cv.wait()

        def group(g, carry):
            b0 = g * LANES
            for j in range(LANES):
                b = b0 + j
                bsplat = jnp.full((LANES,), b, jnp.int32)
                uco = plsc.load_gather(ucol, [bsplat + pbase])
                ico = plsc.load_gather(icol, [bsplat + pbase])
                acc = jnp.zeros((LANES,), jnp.float32)
                for c in range(4):
                    gu = plsc.load_gather(urows, [bsplat, uco + chunks[c]])
                    gv = plsc.load_gather(vrows, [bsplat, ico + chunks[c]])
                    acc += gu * gv * pchunks[c]
                accv[pl.ds(j * LANES, LANES)] = acc
            vec = plsc.load_gather(accv, [col_base])
            for k in range(1, LANES):
                vec += plsc.load_gather(accv, [col_base + k])
            outv[pl.ds(pbase + b0, LANES)] = vec
            return carry

        lax.fori_loop(0, PASS_ROWS // LANES, group, 0)

    do_pass(0, uidxA, iidxA)
    do_pass(PASS_ROWS, uidxB, iidxB)

    pltpu.sync_copy(outv, out_hbm.at[pl.ds(base, B_PER_W)])


@functools.partial(
    pl.kernel,
    mesh=plsc.VectorSubcoreMesh(core_axis_name="c", subcore_axis_name="s"),
    out_type=jax.ShapeDtypeStruct((BATCH,), jnp.float32),
    compiler_params=pltpu.CompilerParams(
        needs_layout_passes=False, use_tc_tiling_on_sc=True),
    scratch_types=[
        pltpu.VMEM((PASS_ROWS,), jnp.int32),
        pltpu.VMEM((PASS_ROWS,), jnp.int32),
        pltpu.VMEM((PASS_ROWS,), jnp.int32),
        pltpu.VMEM((PASS_ROWS,), jnp.int32),
        pltpu.VMEM((B_PER_W,), jnp.int32),
        pltpu.VMEM((B_PER_W,), jnp.int32),
        pltpu.VMEM((PASS_ROWS, PAIR), jnp.float32),
        pltpu.VMEM((PASS_ROWS, PAIR), jnp.float32),
        pltpu.VMEM((RANK,), jnp.float32),
        pltpu.VMEM((B_PER_W,), jnp.float32),
        pltpu.VMEM((LANES * LANES,), jnp.float32),
        pltpu.SemaphoreType.DMA,
        pltpu.SemaphoreType.DMA,
    ],
)
def _sc_kernel(users_hbm, items_hbm, pred_hbm, u2_hbm, v2_hbm, out_hbm,
               uidxA, uidxB, iidxA, iidxB, ucol, icol, urows, vrows,
               pvec, outv, accv, sem_u, sem_v):
    _sc_body(users_hbm, items_hbm, pred_hbm, u2_hbm, v2_hbm, out_hbm,
             uidxA, uidxB, iidxA, iidxB, ucol, icol, urows, vrows,
             pvec, outv, accv, sem_u, sem_v)


def kernel(users, items, embed_U, embed_V, predict_layer):
    pred = predict_layer.reshape(RANK)
    u2 = _tc_pack(embed_U.T)
    v2 = _tc_pack(embed_V.T)
    return _sc_kernel(users, items, pred, u2, v2)
